# baseline (speedup 1.0000x reference)
import functools

import jax
import jax.numpy as jnp
from jax import lax
from jax.experimental import pallas as pl
from jax.experimental.pallas import tpu as pltpu
from jax.experimental.pallas import tpu_sc as plsc

N = 1_000_000
N_DAYS = 2000
N_BUCKETS = 500
N_BASES = 6
Z_BAR = 0.5
BEND_KNOTS = (0.4, 0.5, 0.6, 0.7)

C = 3072
G = C // 16
R = C // 128
NUM_CHUNKS = -(-N // C)
NC = 2
NS = 16
NW = NC * NS

_mesh = plsc.VectorSubcoreMesh(core_axis_name="c", subcore_axis_name="s")


@functools.partial(
    pl.kernel,
    mesh=_mesh,
    compiler_params=pltpu.CompilerParams(
        use_tc_tiling_on_sc=False, needs_layout_passes=False),
    out_type=(
        jax.ShapeDtypeStruct((N,), jnp.float32),
        jax.ShapeDtypeStruct((NC * N, 8), jnp.float32),
    ),
    scratch_types=[
        pltpu.VMEM((2, C), jnp.int32),
        pltpu.VMEM((2, C), jnp.int32),
        pltpu.VMEM((2, C), jnp.float32),
        pltpu.VMEM((2, C), jnp.int32),
        pltpu.VMEM((2, N_BASES, C), jnp.float32),
        pltpu.VMEM((C, 8), jnp.float32),
        pltpu.VMEM((C, 8), jnp.float32),
        pltpu.VMEM((2, C), jnp.float32),
        pltpu.SemaphoreType.DMA,
        pltpu.SemaphoreType.DMA,
        pltpu.SemaphoreType.DMA,
        pltpu.SemaphoreType.DMA,
    ],
)
def _sc_basis_delta(mvoc_hbm, day_hbm, bucket_hbm,
                    w0_hbm, w1_hbm, w2_hbm, w3_hbm, w4_hbm, w5_hbm,
                    out_hbm, tbl_hbm,
                    day_v, bucket_v, mvoc_v, idx_v, pb_v, rows0_v, rows1_v,
                    out_v, sem_in, sem_g0, sem_g1, sem_out):
    w_hbms = (w0_hbm, w1_hbm, w2_hbm, w3_hbm, w4_hbm, w5_hbm)
    rows_refs = (rows0_v, rows1_v)
    cid = lax.axis_index("c")
    sid = lax.axis_index("s")
    wid = sid * NC + cid
    iota = lax.iota(jnp.int32, 16)
    row0 = cid * N

    my_nb = (NUM_CHUNKS - sid + NS - 1) // NS

    def bchunk_base(t):
        return pl.multiple_of(lax.min((sid + t * NS) * C, N - C), 64)

    def bin_copies(t, p):
        b = bchunk_base(t)
        return tuple(
            pltpu.make_async_copy(w_hbms[j].at[pl.ds(b, C)],
                                  pb_v.at[p, j], sem_in)
            for j in range(N_BASES)
        )

    def interleave(p, rows_ref):
        def grp(g, _):
            rid = iota + g * 16
            o = g * 16
            for j in range(N_BASES):
                plsc.store_scatter(rows_ref, [rid, jnp.full((16,), j, jnp.int32)],
                                   pb_v[p, j, pl.ds(o, 16)])
            return 0
        lax.fori_loop(0, G, grp, 0, unroll=2)

    for cp in bin_copies(0, 0):
        cp.start()

    def build_pair(h, _):
        t0 = h * 2

        def one(t, p):
            for cp in bin_copies(t, p):
                cp.wait()

            @pl.when(t + 1 < my_nb)
            def _():
                for cp in bin_copies(t + 1, 1 - p):
                    cp.start()

            @pl.when(t >= 2)
            def _():
                pltpu.make_async_copy(
                    rows_refs[p],
                    tbl_hbm.at[pl.ds(row0 + bchunk_base(t - 2), C)],
                    sem_out,
                ).wait()

            interleave(p, rows_refs[p])
            pltpu.make_async_copy(
                rows_refs[p],
                tbl_hbm.at[pl.ds(row0 + bchunk_base(t), C)],
                sem_out,
            ).start()

        @pl.when(t0 < my_nb)
        def _():
            one(t0, 0)

        @pl.when(t0 + 1 < my_nb)
        def _():
            one(t0 + 1, 1)

        return 0

    lax.fori_loop(0, (my_nb + 1) // 2, build_pair, 0, unroll=False)

    def wait_tbl_write(t):
        par = lax.rem(t, 2)
        for sp in (0, 1):
            @pl.when(par == sp)
            def _():
                pltpu.make_async_copy(
                    rows_refs[sp],
                    tbl_hbm.at[pl.ds(row0 + bchunk_base(t), C)], sem_out,
                ).wait()

    wait_tbl_write(my_nb - 2)
    wait_tbl_write(my_nb - 1)

    plsc.subcore_barrier()

    my_n = (NUM_CHUNKS - wid + NW - 1) // NW

    def chunk_base(t):
        return pl.multiple_of(lax.min((wid + t * NW) * C, N - C), 64)

    def input_copies(t, p):
        b = chunk_base(t)
        return (
            pltpu.make_async_copy(day_hbm.at[pl.ds(b, C)], day_v.at[p], sem_in),
            pltpu.make_async_copy(bucket_hbm.at[pl.ds(b, C)], bucket_v.at[p], sem_in),
            pltpu.make_async_copy(mvoc_hbm.at[pl.ds(b, C)], mvoc_v.at[p], sem_in),
        )

    def build_idx(p):
        def idx_row(r, _):
            for cc in range(8):
                o = (r * 8 + cc) * 16
                d = day_v[p, pl.ds(o, 16)]
                b = bucket_v[p, pl.ds(o, 16)]
                idx_v[p, pl.ds(o, 16)] = d * N_BUCKETS + b + row0
            return 0
        lax.fori_loop(0, R, idx_row, 0, unroll=False)

    def gather_copy(p, sem):
        return pltpu.make_async_copy(
            tbl_hbm.at[idx_v.at[p]], rows_refs[p], sem)

    def compute(t, p):
        rows_ref = rows_refs[p]

        def grp(g, _):
            o = g * 16
            rid = iota + o
            z = jnp.minimum(jnp.maximum(mvoc_v[p, pl.ds(o, 16)], 0.0), 1.0)
            w = [plsc.load_gather(rows_ref, [rid, jnp.full((16,), j, jnp.int32)])
                 for j in range(N_BASES)]
            acc = w[0] + w[1] * (Z_BAR - z)
            for u, bk in enumerate(BEND_KNOTS):
                acc = acc + w[2 + u] * jnp.maximum(bk - z, 0.0)
            out_v[p, pl.ds(o, 16)] = acc
            return 0
        lax.fori_loop(0, G, grp, 0, unroll=2)

    for cp in input_copies(0, 0):
        cp.start()
    for cp in input_copies(0, 0):
        cp.wait()
    build_idx(0)
    gather_copy(0, sem_g0).start()

    @pl.when(my_n > 1)
    def _():
        for cp in input_copies(1, 1):
            cp.start()

    def pair_body(h, _):
        t0 = h * 2

        def one(t, p, sem_cur, sem_nxt):
            q = 1 - p

            @pl.when(t + 1 < my_n)
            def _():
                for cp in input_copies(t + 1, q):
                    cp.wait()
                build_idx(q)
                gather_copy(q, sem_nxt).start()

            gather_copy(p, sem_cur).wait()

            @pl.when(t >= 2)
            def _():
                pltpu.make_async_copy(
                    out_v.at[p], out_hbm.at[pl.ds(chunk_base(t - 2), C)],
                    sem_out,
                ).wait()

            compute(t, p)
            pltpu.make_async_copy(
                out_v.at[p], out_hbm.at[pl.ds(chunk_base(t), C)], sem_out
            ).start()

            @pl.when(t + 2 < my_n)
            def _():
                for cp in input_copies(t + 2, p):
                    cp.start()

        @pl.when(t0 < my_n)
        def _():
            one(t0, 0, sem_g0, sem_g1)

        @pl.when(t0 + 1 < my_n)
        def _():
            one(t0 + 1, 1, sem_g1, sem_g0)

        return 0

    lax.fori_loop(0, (my_n + 1) // 2, pair_body, 0, unroll=False)

    @pl.when(my_n >= 2)
    def _():
        pltpu.make_async_copy(
            out_v.at[lax.rem(my_n - 2, 2)],
            out_hbm.at[pl.ds(chunk_base(my_n - 2), C)], sem_out,
        ).wait()

    pltpu.make_async_copy(
        out_v.at[lax.rem(my_n - 1, 2)],
        out_hbm.at[pl.ds(chunk_base(my_n - 1), C)], sem_out,
    ).wait()


def kernel(mvoc, day_idx, bucket_idx, weights):
    wt = weights.transpose(2, 0, 1)
    planes = [wt[j].reshape(-1) for j in range(N_BASES)]
    out, _ = _sc_basis_delta(mvoc, day_idx.astype(jnp.int32),
                             bucket_idx.astype(jnp.int32), *planes)
    return out.reshape(N, 1)


# speedup vs baseline: 1.0086x; 1.0086x over previous
import functools

import jax
import jax.numpy as jnp
from jax import lax
from jax.experimental import pallas as pl
from jax.experimental.pallas import tpu as pltpu
from jax.experimental.pallas import tpu_sc as plsc

N = 1_000_000
N_DAYS = 2000
N_BUCKETS = 500
N_BASES = 6
Z_BAR = 0.5
BEND_KNOTS = (0.4, 0.5, 0.6, 0.7)

C = 2048
G = C // 16
R = C // 128
NUM_CHUNKS = -(-N // C)
NC = 2
NS = 16
NW = NC * NS

_mesh = plsc.VectorSubcoreMesh(core_axis_name="c", subcore_axis_name="s")


@functools.partial(
    pl.kernel,
    mesh=_mesh,
    compiler_params=pltpu.CompilerParams(
        use_tc_tiling_on_sc=False, needs_layout_passes=False),
    out_type=(
        jax.ShapeDtypeStruct((N,), jnp.float32),
        jax.ShapeDtypeStruct((NC * N, 8), jnp.float32),
    ),
    scratch_types=[
        pltpu.VMEM((2, C), jnp.int32),
        pltpu.VMEM((2, C), jnp.int32),
        pltpu.VMEM((2, C), jnp.float32),
        pltpu.VMEM((2, C), jnp.int32),
        pltpu.VMEM((2, N_BASES, C), jnp.float32),
        pltpu.VMEM((C, 8), jnp.float32),
        pltpu.VMEM((C, 8), jnp.float32),
        pltpu.VMEM((2, C), jnp.float32),
        pltpu.SemaphoreType.DMA,
        pltpu.SemaphoreType.DMA,
        pltpu.SemaphoreType.DMA,
        pltpu.SemaphoreType.DMA,
    ],
)
def _sc_basis_delta(mvoc_hbm, day_hbm, bucket_hbm,
                    w0_hbm, w1_hbm, w2_hbm, w3_hbm, w4_hbm, w5_hbm,
                    out_hbm, tbl_hbm,
                    day_v, bucket_v, mvoc_v, idx_v, pb_v, rows0_v, rows1_v,
                    out_v, sem_in, sem_g0, sem_g1, sem_out):
    w_hbms = (w0_hbm, w1_hbm, w2_hbm, w3_hbm, w4_hbm, w5_hbm)
    rows_refs = (rows0_v, rows1_v)
    cid = lax.axis_index("c")
    sid = lax.axis_index("s")
    wid = sid * NC + cid
    iota = lax.iota(jnp.int32, 16)
    row0 = cid * N

    my_nb = (NUM_CHUNKS - sid + NS - 1) // NS

    def bchunk_base(t):
        return pl.multiple_of(lax.min((sid + t * NS) * C, N - C), 64)

    def bin_copies(t, p):
        b = bchunk_base(t)
        return tuple(
            pltpu.make_async_copy(w_hbms[j].at[pl.ds(b, C)],
                                  pb_v.at[p, j], sem_in)
            for j in range(N_BASES)
        )

    def interleave(p, rows_ref):
        def grp(g, _):
            rid = iota + g * 16
            o = g * 16
            for j in range(N_BASES):
                plsc.store_scatter(rows_ref, [rid, jnp.full((16,), j, jnp.int32)],
                                   pb_v[p, j, pl.ds(o, 16)])
            return 0
        lax.fori_loop(0, G, grp, 0, unroll=4)

    for cp in bin_copies(0, 0):
        cp.start()

    def build_pair(h, _):
        t0 = h * 2

        def one(t, p):
            for cp in bin_copies(t, p):
                cp.wait()

            @pl.when(t + 1 < my_nb)
            def _():
                for cp in bin_copies(t + 1, 1 - p):
                    cp.start()

            @pl.when(t >= 2)
            def _():
                pltpu.make_async_copy(
                    rows_refs[p],
                    tbl_hbm.at[pl.ds(row0 + bchunk_base(t - 2), C)],
                    sem_out,
                ).wait()

            interleave(p, rows_refs[p])
            pltpu.make_async_copy(
                rows_refs[p],
                tbl_hbm.at[pl.ds(row0 + bchunk_base(t), C)],
                sem_out,
            ).start()

        @pl.when(t0 < my_nb)
        def _():
            one(t0, 0)

        @pl.when(t0 + 1 < my_nb)
        def _():
            one(t0 + 1, 1)

        return 0

    lax.fori_loop(0, (my_nb + 1) // 2, build_pair, 0, unroll=False)

    def wait_tbl_write(t):
        par = lax.rem(t, 2)
        for sp in (0, 1):
            @pl.when(par == sp)
            def _():
                pltpu.make_async_copy(
                    rows_refs[sp],
                    tbl_hbm.at[pl.ds(row0 + bchunk_base(t), C)], sem_out,
                ).wait()

    wait_tbl_write(my_nb - 2)
    wait_tbl_write(my_nb - 1)

    plsc.subcore_barrier()

    my_n = (NUM_CHUNKS - wid + NW - 1) // NW

    def chunk_base(t):
        return pl.multiple_of(lax.min((wid + t * NW) * C, N - C), 64)

    def input_copies(t, p):
        b = chunk_base(t)
        return (
            pltpu.make_async_copy(day_hbm.at[pl.ds(b, C)], day_v.at[p], sem_in),
            pltpu.make_async_copy(bucket_hbm.at[pl.ds(b, C)], bucket_v.at[p], sem_in),
            pltpu.make_async_copy(mvoc_hbm.at[pl.ds(b, C)], mvoc_v.at[p], sem_in),
        )

    def build_idx(p):
        def idx_row(r, _):
            for cc in range(8):
                o = (r * 8 + cc) * 16
                d = day_v[p, pl.ds(o, 16)]
                b = bucket_v[p, pl.ds(o, 16)]
                idx_v[p, pl.ds(o, 16)] = d * N_BUCKETS + b + row0
            return 0
        lax.fori_loop(0, R, idx_row, 0, unroll=False)

    def gather_copy(p, sem):
        return pltpu.make_async_copy(
            tbl_hbm.at[idx_v.at[p]], rows_refs[p], sem)

    def compute(t, p):
        rows_ref = rows_refs[p]

        def grp(g, _):
            o = g * 16
            rid = iota + o
            z = jnp.minimum(jnp.maximum(mvoc_v[p, pl.ds(o, 16)], 0.0), 1.0)
            w = [plsc.load_gather(rows_ref, [rid, jnp.full((16,), j, jnp.int32)])
                 for j in range(N_BASES)]
            acc = w[0] + w[1] * (Z_BAR - z)
            for u, bk in enumerate(BEND_KNOTS):
                acc = acc + w[2 + u] * jnp.maximum(bk - z, 0.0)
            out_v[p, pl.ds(o, 16)] = acc
            return 0
        lax.fori_loop(0, G, grp, 0, unroll=2)

    for cp in input_copies(0, 0):
        cp.start()
    for cp in input_copies(0, 0):
        cp.wait()
    build_idx(0)
    gather_copy(0, sem_g0).start()

    @pl.when(my_n > 1)
    def _():
        for cp in input_copies(1, 1):
            cp.start()

    def pair_body(h, _):
        t0 = h * 2

        def one(t, p, sem_cur, sem_nxt):
            q = 1 - p

            @pl.when(t + 1 < my_n)
            def _():
                for cp in input_copies(t + 1, q):
                    cp.wait()
                build_idx(q)
                gather_copy(q, sem_nxt).start()

            gather_copy(p, sem_cur).wait()

            @pl.when(t >= 2)
            def _():
                pltpu.make_async_copy(
                    out_v.at[p], out_hbm.at[pl.ds(chunk_base(t - 2), C)],
                    sem_out,
                ).wait()

            compute(t, p)
            pltpu.make_async_copy(
                out_v.at[p], out_hbm.at[pl.ds(chunk_base(t), C)], sem_out
            ).start()

            @pl.when(t + 2 < my_n)
            def _():
                for cp in input_copies(t + 2, p):
                    cp.start()

        @pl.when(t0 < my_n)
        def _():
            one(t0, 0, sem_g0, sem_g1)

        @pl.when(t0 + 1 < my_n)
        def _():
            one(t0 + 1, 1, sem_g1, sem_g0)

        return 0

    lax.fori_loop(0, (my_n + 1) // 2, pair_body, 0, unroll=False)

    @pl.when(my_n >= 2)
    def _():
        pltpu.make_async_copy(
            out_v.at[lax.rem(my_n - 2, 2)],
            out_hbm.at[pl.ds(chunk_base(my_n - 2), C)], sem_out,
        ).wait()

    pltpu.make_async_copy(
        out_v.at[lax.rem(my_n - 1, 2)],
        out_hbm.at[pl.ds(chunk_base(my_n - 1), C)], sem_out,
    ).wait()


def kernel(mvoc, day_idx, bucket_idx, weights):
    wt = weights.transpose(2, 0, 1)
    planes = [wt[j].reshape(-1) for j in range(N_BASES)]
    out, _ = _sc_basis_delta(mvoc, day_idx.astype(jnp.int32),
                             bucket_idx.astype(jnp.int32), *planes)
    return out.reshape(N, 1)
